# baseline (speedup 1.0000x reference)
import functools

import jax
import jax.numpy as jnp
from jax import lax
from jax.experimental import pallas as pl
from jax.experimental.pallas import tpu as pltpu
from jax.experimental.pallas import tpu_sc as plsc

_N = 10000
_E = 320000
_D = 128
_H = 64
_HH = _H // 2
_H4 = _H // 4

_NS = 16
_K = 80
_EPT = _E // _NS
_NCH = _EPT // _K
_NP = 10240
_ZR = _NP // _NS
_G = 2

_NB1 = 5
_RB1 = _N // 2 // _NB1
_NB3 = 4
_RB3 = _NP // 2 // _NB3


def _stage1(x2, Wbd, b2):
  def body(x_ref, w_ref, b_ref, o_ref):
    o_ref[...] = jnp.maximum(
        jnp.dot(x_ref[...], w_ref[...], preferred_element_type=jnp.float32)
        + b_ref[...], 0.0)

  return pl.pallas_call(
      body,
      grid=(_NB1,),
      in_specs=[
          pl.BlockSpec((_RB1, 2 * _D), lambda j: (j, 0)),
          pl.BlockSpec((2 * _D, 2 * _H), lambda j: (0, 0)),
          pl.BlockSpec((1, 2 * _H), lambda j: (0, 0)),
      ],
      out_specs=pl.BlockSpec((_RB1, 2 * _H), lambda j: (j, 0)),
      out_shape=jax.ShapeDtypeStruct((_N // 2, 2 * _H), jnp.float32),
  )(x2, Wbd, b2)


def _sc_scatter(h1, src1, dst1, h2, src2, dst2, z_h, z_c, ones_g):
  mesh = plsc.VectorSubcoreMesh(core_axis_name="c", subcore_axis_name="s")

  @functools.partial(
      pl.kernel,
      out_type=[
          jax.ShapeDtypeStruct((2 * _NP, _H), jnp.float32),
          jax.ShapeDtypeStruct((2 * _NP, _G), jnp.float32),
      ],
      mesh=mesh,
      scratch_types=[
          pltpu.VMEM((_EPT,), jnp.int32),
          pltpu.VMEM((_EPT,), jnp.int32),
          pltpu.VMEM((_K, _H), jnp.float32),
          pltpu.VMEM((_K, _H), jnp.float32),
          pltpu.VMEM((_K, _G), jnp.float32),
          pltpu.VMEM_SHARED((_NP, _H), jnp.float32),
          pltpu.VMEM_SHARED((_NP, _G), jnp.float32),
          pltpu.SemaphoreType.DMA,
          pltpu.SemaphoreType.DMA,
      ],
      compiler_params=pltpu.CompilerParams(use_tc_tiling_on_sc=False),
  )
  def k(h1_hbm, s1_hbm, d1_hbm, h2_hbm, s2_hbm, d2_hbm,
        zh_hbm, zc_hbm, ones_hbm, a_out, c_out,
        src_v, dst_v, msg0, msg1, ones_v, a_s, c_s, sem0, sem1):
    cid = lax.axis_index("c")
    sid = lax.axis_index("s")
    pltpu.sync_copy(zh_hbm, a_s.at[pl.ds(sid * _ZR, _ZR)])
    pltpu.sync_copy(zc_hbm, c_s.at[pl.ds(sid * _ZR, _ZR)])
    pltpu.sync_copy(ones_hbm, ones_v)
    e0 = sid * _EPT

    @pl.when(cid == 0)
    def _():
      pltpu.sync_copy(s1_hbm.at[pl.ds(e0, _EPT)], src_v)
      pltpu.sync_copy(d1_hbm.at[pl.ds(e0, _EPT)], dst_v)

    @pl.when(cid == 1)
    def _():
      pltpu.sync_copy(s2_hbm.at[pl.ds(e0, _EPT)], src_v)
      pltpu.sync_copy(d2_hbm.at[pl.ds(e0, _EPT)], dst_v)

    plsc.subcore_barrier()

    def edge_loop(h_hbm):
      msgs = (msg0, msg1)
      sems = (sem0, sem1)
      pltpu.async_copy(h_hbm.at[src_v.at[pl.ds(0, _K)]], msg0, sem0)
      pltpu.async_copy(h_hbm.at[src_v.at[pl.ds(_K, _K)]], msg1, sem1)

      @pl.loop(0, _NCH, step=2)
      def _(j):
        for b in range(2):
          jj = j + b
          pltpu.make_async_copy(h_hbm.at[src_v.at[pl.ds(jj * _K, _K)]],
                                msgs[b], sems[b]).wait()
          pltpu.sync_copy(msgs[b], a_s.at[dst_v.at[pl.ds(jj * _K, _K)]],
                          add=True)
          pltpu.sync_copy(ones_v, c_s.at[src_v.at[pl.ds(jj * _K, _K)]],
                          add=True)

          @pl.when(jj + 2 < _NCH)
          def _():
            pltpu.async_copy(h_hbm.at[src_v.at[pl.ds((jj + 2) * _K, _K)]],
                             msgs[b], sems[b])

    @pl.when(cid == 0)
    def _():
      edge_loop(h1_hbm)

    @pl.when(cid == 1)
    def _():
      edge_loop(h2_hbm)

    plsc.subcore_barrier()
    row = pl.ds(sid * _ZR, _ZR)
    orow = pl.ds(cid * _NP + sid * _ZR, _ZR)
    pltpu.sync_copy(a_s.at[row], a_out.at[orow])
    pltpu.sync_copy(c_s.at[row], c_out.at[orow])

  return k(h1, src1, dst1, h2, src2, dst2, z_h, z_c, ones_g)


def _stage3(a2, c2, Wbd, bb2, Wd, bd, Wo, bo):
  def body(a_ref, c_ref, wb, bbr, wd, bdr, wo, bor, o_ref, acc0, acc1):
    g = pl.program_id(0)
    j = pl.program_id(1)
    y = jnp.dot(a_ref[...], wb[0], preferred_element_type=jnp.float32) + bbr[0]
    y = jnp.maximum(y, 0.0)
    part = (jnp.sum(y[:, :_HH] * c_ref[:, 0:1], axis=0, keepdims=True)
            + jnp.sum(y[:, _HH:] * c_ref[:, _G:_G + 1], axis=0,
                      keepdims=True))

    @pl.when((g == 0) & (j == 0))
    def _():
      acc0[...] = part

    @pl.when((g == 0) & (j > 0))
    def _():
      acc0[...] = acc0[...] + part

    @pl.when((g == 1) & (j == 0))
    def _():
      acc1[...] = part

    @pl.when((g == 1) & (j > 0))
    def _():
      acc1[...] = acc1[...] + part

    @pl.when((g == 1) & (j == _NB3 - 1))
    def _():
      p0 = acc0[...] * (1.0 / _N)
      p1 = acc1[...] * (1.0 / _N)
      t = (jnp.dot(p0, wd[0:_HH], preferred_element_type=jnp.float32)
           + jnp.dot(p1, wd[_HH:_H], preferred_element_type=jnp.float32)
           + bdr[...])
      t = jnp.maximum(t, 0.0)
      z = jnp.dot(t, wo[...], preferred_element_type=jnp.float32) + bor[...]
      o_ref[...] = 1.0 / (1.0 + jnp.exp(-z))

  return pl.pallas_call(
      body,
      grid=(2, _NB3),
      in_specs=[
          pl.BlockSpec((_RB3, 2 * _H), lambda g, j: (g * _NB3 + j, 0)),
          pl.BlockSpec((_RB3, 2 * _G), lambda g, j: (g * _NB3 + j, 0)),
          pl.BlockSpec((1, 2 * _H, 2 * _HH), lambda g, j: (g, 0, 0)),
          pl.BlockSpec((1, 1, 2 * _HH), lambda g, j: (g, 0, 0)),
          pl.BlockSpec((_H, _H4), lambda g, j: (0, 0)),
          pl.BlockSpec((1, _H4), lambda g, j: (0, 0)),
          pl.BlockSpec((_H4, 1), lambda g, j: (0, 0)),
          pl.BlockSpec((1, 1), lambda g, j: (0, 0)),
      ],
      out_specs=pl.BlockSpec((1, 1), lambda g, j: (0, 0)),
      out_shape=jax.ShapeDtypeStruct((1, 1), jnp.float32),
      scratch_shapes=[
          pltpu.VMEM((1, _HH), jnp.float32),
          pltpu.VMEM((1, _HH), jnp.float32),
      ],
  )(a2, c2, Wbd, bb2, Wd, bd, Wo, bo)


def _blockdiag(W):
  din, dout = W.shape
  z = jnp.zeros((din, dout), W.dtype)
  return jnp.concatenate(
      [jnp.concatenate([W, z], axis=1), jnp.concatenate([z, W], axis=1)],
      axis=0)


def kernel(x1, edge_index1, x2, edge_index2, W1a, b1a, W1b, b1b,
           W2a, b2a, W2b, b2b, Wd, bd, Wo, bo):
  src1 = edge_index1[0]
  dst1 = edge_index1[1]
  src2 = edge_index2[0]
  dst2 = edge_index2[1]

  z_h = jnp.zeros((_ZR, _H), jnp.float32)
  z_c = jnp.zeros((_ZR, _G), jnp.float32)
  ones_g = jnp.ones((_K, _G), jnp.float32)

  h1 = _stage1(x1.reshape(_N // 2, 2 * _D), _blockdiag(W1a),
               jnp.tile(b1a, 2).reshape(1, 2 * _H))
  h2 = _stage1(x2.reshape(_N // 2, 2 * _D), _blockdiag(W2a),
               jnp.tile(b2a, 2).reshape(1, 2 * _H))

  a_out, c_out = _sc_scatter(
      h1.reshape(_N, _H), src1, dst1, h2.reshape(_N, _H), src2, dst2,
      z_h, z_c, ones_g)
  a_cat = a_out.reshape(_NP, 2 * _H)
  c_cat = c_out.reshape(_NP, 2 * _G)

  Wbd = jnp.stack([_blockdiag(W1b), _blockdiag(W2b)])
  bb2 = jnp.stack([jnp.tile(b1b, 2), jnp.tile(b2b, 2)]).reshape(2, 1, 2 * _HH)
  return _stage3(a_cat, c_cat, Wbd, bb2, Wd, bd.reshape(1, _H4),
                 Wo.reshape(_H4, 1), bo.reshape(1, 1))


# speedup vs baseline: 1.2210x; 1.2210x over previous
import functools

import jax
import jax.numpy as jnp
from jax import lax
from jax.experimental import pallas as pl
from jax.experimental.pallas import tpu as pltpu
from jax.experimental.pallas import tpu_sc as plsc

_N = 10000
_E = 320000
_D = 128
_H = 64
_HH = _H // 2
_H4 = _H // 4

_NS = 16
_K = 200
_EPT = _E // _NS
_NCH = _EPT // _K
_KC = 2000
_NCC = _EPT // _KC
_NP = 10240
_ZR = _NP // _NS
_G = 2

_NB1 = 5
_RB1 = _N // 2 // _NB1
_NB3 = 4
_RB3 = _NP // 2 // _NB3
_CB3 = _RB3 * 2 * _G // 128


def _stage1(x2, Wbd, b2):
  def body(x_ref, w_ref, b_ref, o_ref):
    o_ref[...] = jnp.maximum(
        jnp.dot(x_ref[...], w_ref[...], preferred_element_type=jnp.float32)
        + b_ref[...], 0.0)

  return pl.pallas_call(
      body,
      grid=(_NB1,),
      in_specs=[
          pl.BlockSpec((_RB1, 2 * _D), lambda j: (j, 0)),
          pl.BlockSpec((2 * _D, 2 * _H), lambda j: (0, 0)),
          pl.BlockSpec((1, 2 * _H), lambda j: (0, 0)),
      ],
      out_specs=pl.BlockSpec((_RB1, 2 * _H), lambda j: (j, 0)),
      out_shape=jax.ShapeDtypeStruct((_N // 2, 2 * _H), jnp.float32),
  )(x2, Wbd, b2)


def _sc_scatter(h1, src1, dst1, h2, src2, dst2, z_h, z_c, ones_g):
  mesh = plsc.VectorSubcoreMesh(core_axis_name="c", subcore_axis_name="s")

  @functools.partial(
      pl.kernel,
      out_type=[
          jax.ShapeDtypeStruct((2 * _NP, _H), jnp.float32),
          jax.ShapeDtypeStruct((2 * _NP, _G), jnp.float32),
      ],
      mesh=mesh,
      scratch_types=[
          pltpu.VMEM((_EPT,), jnp.int32),
          pltpu.VMEM((_EPT,), jnp.int32),
          pltpu.VMEM((_K, _H), jnp.float32),
          pltpu.VMEM((_K, _H), jnp.float32),
          pltpu.VMEM((_KC, _G), jnp.float32),
          pltpu.VMEM_SHARED((_NP, _H), jnp.float32),
          pltpu.VMEM_SHARED((_NP, _G), jnp.float32),
          pltpu.SemaphoreType.DMA,
          pltpu.SemaphoreType.DMA,
      ],
      compiler_params=pltpu.CompilerParams(use_tc_tiling_on_sc=False),
  )
  def k(h1_hbm, s1_hbm, d1_hbm, h2_hbm, s2_hbm, d2_hbm,
        zh_hbm, zc_hbm, ones_hbm, a_out, c_out,
        src_v, dst_v, msg0, msg1, ones_v, a_s, c_s, sem0, sem1):
    cid = lax.axis_index("c")
    sid = lax.axis_index("s")
    pltpu.sync_copy(zh_hbm, a_s.at[pl.ds(sid * _ZR, _ZR)])
    pltpu.sync_copy(zc_hbm, c_s.at[pl.ds(sid * _ZR, _ZR)])
    pltpu.sync_copy(ones_hbm, ones_v)
    e0 = sid * _EPT

    @pl.when(cid == 0)
    def _():
      pltpu.sync_copy(s1_hbm.at[pl.ds(e0, _EPT)], src_v)
      pltpu.sync_copy(d1_hbm.at[pl.ds(e0, _EPT)], dst_v)

    @pl.when(cid == 1)
    def _():
      pltpu.sync_copy(s2_hbm.at[pl.ds(e0, _EPT)], src_v)
      pltpu.sync_copy(d2_hbm.at[pl.ds(e0, _EPT)], dst_v)

    plsc.subcore_barrier()

    def edge_loop(h_hbm):
      msgs = (msg0, msg1)
      sems = (sem0, sem1)
      pltpu.async_copy(h_hbm.at[src_v.at[pl.ds(0, _K)]], msg0, sem0)
      pltpu.async_copy(h_hbm.at[src_v.at[pl.ds(_K, _K)]], msg1, sem1)

      @pl.loop(0, _NCH, step=2)
      def _(j):
        for b in range(2):
          jj = j + b
          pltpu.make_async_copy(h_hbm.at[src_v.at[pl.ds(jj * _K, _K)]],
                                msgs[b], sems[b]).wait()
          pltpu.sync_copy(msgs[b], a_s.at[dst_v.at[pl.ds(jj * _K, _K)]],
                          add=True)

          @pl.when(jj + 2 < _NCH)
          def _():
            pltpu.async_copy(h_hbm.at[src_v.at[pl.ds((jj + 2) * _K, _K)]],
                             msgs[b], sems[b])

      @pl.loop(0, _NCC)
      def _(j):
        pltpu.sync_copy(ones_v, c_s.at[src_v.at[pl.ds(j * _KC, _KC)]],
                        add=True)

    @pl.when(cid == 0)
    def _():
      edge_loop(h1_hbm)

    @pl.when(cid == 1)
    def _():
      edge_loop(h2_hbm)

    plsc.subcore_barrier()
    row = pl.ds(sid * _ZR, _ZR)
    orow = pl.ds(cid * _NP + sid * _ZR, _ZR)
    pltpu.sync_copy(a_s.at[row], a_out.at[orow])
    pltpu.sync_copy(c_s.at[row], c_out.at[orow])

  return k(h1, src1, dst1, h2, src2, dst2, z_h, z_c, ones_g)


def _stage3(a2, c2, Wbd, bb2, Wd, bd, Wo, bo):
  def body(a_ref, c_ref, wb, bbr, wd, bdr, wo, bor, o_ref, acc0, acc1):
    g = pl.program_id(0)
    j = pl.program_id(1)
    y = jnp.dot(a_ref[...], wb[0], preferred_element_type=jnp.float32) + bbr[0]
    y = jnp.maximum(y, 0.0)
    part = (jnp.sum(y[:, :_HH] * c_ref[:, 0:1], axis=0, keepdims=True)
            + jnp.sum(y[:, _HH:] * c_ref[:, _G:_G + 1], axis=0,
                      keepdims=True))

    @pl.when((g == 0) & (j == 0))
    def _():
      acc0[...] = part

    @pl.when((g == 0) & (j > 0))
    def _():
      acc0[...] = acc0[...] + part

    @pl.when((g == 1) & (j == 0))
    def _():
      acc1[...] = part

    @pl.when((g == 1) & (j > 0))
    def _():
      acc1[...] = acc1[...] + part

    @pl.when((g == 1) & (j == _NB3 - 1))
    def _():
      p0 = acc0[...] * (1.0 / _N)
      p1 = acc1[...] * (1.0 / _N)
      t = (jnp.dot(p0, wd[0:_HH], preferred_element_type=jnp.float32)
           + jnp.dot(p1, wd[_HH:_H], preferred_element_type=jnp.float32)
           + bdr[...])
      t = jnp.maximum(t, 0.0)
      z = jnp.dot(t, wo[...], preferred_element_type=jnp.float32) + bor[...]
      o_ref[...] = 1.0 / (1.0 + jnp.exp(-z))

  return pl.pallas_call(
      body,
      grid=(2, _NB3),
      in_specs=[
          pl.BlockSpec((_RB3, 2 * _H), lambda g, j: (g * _NB3 + j, 0)),
          pl.BlockSpec((_RB3, 2 * _G), lambda g, j: (g * _NB3 + j, 0)),
          pl.BlockSpec((1, 2 * _H, 2 * _HH), lambda g, j: (g, 0, 0)),
          pl.BlockSpec((1, 1, 2 * _HH), lambda g, j: (g, 0, 0)),
          pl.BlockSpec((_H, _H4), lambda g, j: (0, 0)),
          pl.BlockSpec((1, _H4), lambda g, j: (0, 0)),
          pl.BlockSpec((_H4, 1), lambda g, j: (0, 0)),
          pl.BlockSpec((1, 1), lambda g, j: (0, 0)),
      ],
      out_specs=pl.BlockSpec((1, 1), lambda g, j: (0, 0)),
      out_shape=jax.ShapeDtypeStruct((1, 1), jnp.float32),
      scratch_shapes=[
          pltpu.VMEM((1, _HH), jnp.float32),
          pltpu.VMEM((1, _HH), jnp.float32),
      ],
  )(a2, c2, Wbd, bb2, Wd, bd, Wo, bo)


def _blockdiag(W):
  din, dout = W.shape
  z = jnp.zeros((din, dout), W.dtype)
  return jnp.concatenate(
      [jnp.concatenate([W, z], axis=1), jnp.concatenate([z, W], axis=1)],
      axis=0)


def kernel(x1, edge_index1, x2, edge_index2, W1a, b1a, W1b, b1b,
           W2a, b2a, W2b, b2b, Wd, bd, Wo, bo):
  src1 = edge_index1[0]
  dst1 = edge_index1[1]
  src2 = edge_index2[0]
  dst2 = edge_index2[1]

  z_h = jnp.zeros((_ZR, _H), jnp.float32)
  z_c = jnp.zeros((_ZR, _G), jnp.float32)
  ones_g = jnp.ones((_KC, _G), jnp.float32)

  h1 = _stage1(x1.reshape(_N // 2, 2 * _D), _blockdiag(W1a),
               jnp.tile(b1a, 2).reshape(1, 2 * _H))
  h2 = _stage1(x2.reshape(_N // 2, 2 * _D), _blockdiag(W2a),
               jnp.tile(b2a, 2).reshape(1, 2 * _H))

  a_out, c_out = _sc_scatter(
      h1.reshape(_N, _H), src1, dst1, h2.reshape(_N, _H), src2, dst2,
      z_h, z_c, ones_g)
  a2 = a_out.reshape(_NP, 2 * _H)
  c2 = c_out.reshape(_NP, 2 * _G)

  Wbd = jnp.stack([_blockdiag(W1b), _blockdiag(W2b)])
  bb2 = jnp.stack([jnp.tile(b1b, 2), jnp.tile(b2b, 2)]).reshape(2, 1, 2 * _HH)
  return _stage3(a2, c2, Wbd, bb2, Wd, bd.reshape(1, _H4),
                 Wo.reshape(_H4, 1), bo.reshape(1, 1))


# speedup vs baseline: 1.2363x; 1.0125x over previous
import functools

import jax
import jax.numpy as jnp
from jax import lax
from jax.experimental import pallas as pl
from jax.experimental.pallas import tpu as pltpu
from jax.experimental.pallas import tpu_sc as plsc

_N = 10000
_E = 320000
_D = 128
_H = 64
_HH = _H // 2
_H4 = _H // 4

_NW = 32
_NS = 16
_K = 200
_EPT = _E // _NW
_NCH = _EPT // _K
_KC = 2000
_NCC = _EPT // _KC
_NP = 10240
_ZR = _NP // _NS
_G = 2

_NB1 = 5
_RB1 = _N // 2 // _NB1
_EB1 = _E // _NB1
_NB3 = 4
_RB3 = _NP // 2 // _NB3


def _stage1(x2, Wbd, b2, eidx):
  def body(x_ref, w_ref, b_ref, e_ref, h_ref, s_ref, d_ref):
    h_ref[...] = jnp.maximum(
        jnp.dot(x_ref[...], w_ref[...], preferred_element_type=jnp.float32)
        + b_ref[...], 0.0)

    @pl.when(pl.program_id(0) == 0)
    def _():
      s_ref[...] = e_ref[0]
      d_ref[...] = e_ref[1]

  return pl.pallas_call(
      body,
      grid=(_NB1,),
      in_specs=[
          pl.BlockSpec((_RB1, 2 * _D), lambda j: (j, 0)),
          pl.BlockSpec((2 * _D, 2 * _H), lambda j: (0, 0)),
          pl.BlockSpec((1, 2 * _H), lambda j: (0, 0)),
          pl.BlockSpec((2, _E), lambda j: (0, 0)),
      ],
      out_specs=[
          pl.BlockSpec((_RB1, 2 * _H), lambda j: (j, 0)),
          pl.BlockSpec((_E,), lambda j: (0,)),
          pl.BlockSpec((_E,), lambda j: (0,)),
      ],
      out_shape=[
          jax.ShapeDtypeStruct((_N // 2, 2 * _H), jnp.float32),
          jax.ShapeDtypeStruct((_E,), jnp.int32),
          jax.ShapeDtypeStruct((_E,), jnp.int32),
      ],
  )(x2, Wbd, b2, eidx)


def _sc_scatter(h, src, dst, z_h, z_c, ones_g, dep=None):
  mesh = plsc.VectorSubcoreMesh(core_axis_name="c", subcore_axis_name="s")

  @functools.partial(
      pl.kernel,
      out_type=[
          jax.ShapeDtypeStruct((2 * _NP, _H), jnp.float32),
          jax.ShapeDtypeStruct((2 * _NP, _G), jnp.float32),
      ],
      mesh=mesh,
      scratch_types=[
          pltpu.VMEM((_EPT,), jnp.int32),
          pltpu.VMEM((_EPT,), jnp.int32),
          pltpu.VMEM((_K, _H), jnp.float32),
          pltpu.VMEM((_K, _H), jnp.float32),
          pltpu.VMEM((_KC, _G), jnp.float32),
          pltpu.VMEM_SHARED((_NP, _H), jnp.float32),
          pltpu.VMEM_SHARED((_NP, _G), jnp.float32),
          pltpu.SemaphoreType.DMA,
          pltpu.SemaphoreType.DMA,
      ],
      compiler_params=pltpu.CompilerParams(use_tc_tiling_on_sc=False),
  )
  def k(h_hbm, s_hbm, d_hbm, zh_hbm, zc_hbm, ones_hbm, dep_hbm, a_out,
        c_out, src_v, dst_v, msg0, msg1, ones_v, a_s, c_s, sem0, sem1):
    del dep_hbm
    cid = lax.axis_index("c")
    sid = lax.axis_index("s")
    wid = cid * _NS + sid
    pltpu.sync_copy(zh_hbm, a_s.at[pl.ds(sid * _ZR, _ZR)])
    pltpu.sync_copy(zc_hbm, c_s.at[pl.ds(sid * _ZR, _ZR)])
    pltpu.sync_copy(ones_hbm, ones_v)
    e0 = wid * _EPT
    pltpu.sync_copy(s_hbm.at[pl.ds(e0, _EPT)], src_v)
    pltpu.sync_copy(d_hbm.at[pl.ds(e0, _EPT)], dst_v)
    plsc.subcore_barrier()

    msgs = (msg0, msg1)
    sems = (sem0, sem1)
    pltpu.async_copy(h_hbm.at[src_v.at[pl.ds(0, _K)]], msg0, sem0)
    pltpu.async_copy(h_hbm.at[src_v.at[pl.ds(_K, _K)]], msg1, sem1)

    @pl.loop(0, _NCH, step=2)
    def _(j):
      for b in range(2):
        jj = j + b
        pltpu.make_async_copy(h_hbm.at[src_v.at[pl.ds(jj * _K, _K)]],
                              msgs[b], sems[b]).wait()
        pltpu.sync_copy(msgs[b], a_s.at[dst_v.at[pl.ds(jj * _K, _K)]],
                        add=True)

        @pl.when(jj + 2 < _NCH)
        def _():
          pltpu.async_copy(h_hbm.at[src_v.at[pl.ds((jj + 2) * _K, _K)]],
                           msgs[b], sems[b])

    @pl.loop(0, _NCC)
    def _(j):
      pltpu.sync_copy(ones_v, c_s.at[src_v.at[pl.ds(j * _KC, _KC)]],
                      add=True)

    plsc.subcore_barrier()
    row = pl.ds(sid * _ZR, _ZR)
    orow = pl.ds(cid * _NP + sid * _ZR, _ZR)
    pltpu.sync_copy(a_s.at[row], a_out.at[orow])
    pltpu.sync_copy(c_s.at[row], c_out.at[orow])

  if dep is None:
    dep = z_c
  return k(h, src, dst, z_h, z_c, ones_g, dep)


def _stage3(a1, c1, a2, c2, Wbd1, Wbd2, bb1, bb2, Wd, bd, Wo, bo):
  half = _NP // 2 // _RB3

  def body(a1c0, a1c1, c1c0, c1c1, a2c0, a2c1, c2c0, c2c1,
           wb1, wb2, b1r, b2r, wd, bdr, wo, bor, o_ref, acc0, acc1):
    j = pl.program_id(0)

    def weighted(ac0, ac1, cc0, cc1, wb, br):
      y = jnp.dot(ac0[...] + ac1[...], wb[...],
                  preferred_element_type=jnp.float32) + br[...]
      y = jnp.maximum(y, 0.0)
      ce = cc0[:, 0:1] + cc1[:, 0:1]
      co = cc0[:, _G:_G + 1] + cc1[:, _G:_G + 1]
      return (jnp.sum(y[:, :_HH] * ce, axis=0, keepdims=True)
              + jnp.sum(y[:, _HH:] * co, axis=0, keepdims=True))

    p1 = weighted(a1c0, a1c1, c1c0, c1c1, wb1, b1r)
    p2 = weighted(a2c0, a2c1, c2c0, c2c1, wb2, b2r)

    @pl.when(j == 0)
    def _():
      acc0[...] = p1
      acc1[...] = p2

    @pl.when(j > 0)
    def _():
      acc0[...] = acc0[...] + p1
      acc1[...] = acc1[...] + p2

    @pl.when(j == _NB3 - 1)
    def _():
      p0 = acc0[...] * (1.0 / _N)
      p1f = acc1[...] * (1.0 / _N)
      t = (jnp.dot(p0, wd[0:_HH], preferred_element_type=jnp.float32)
           + jnp.dot(p1f, wd[_HH:_H], preferred_element_type=jnp.float32)
           + bdr[...])
      t = jnp.maximum(t, 0.0)
      z = jnp.dot(t, wo[...], preferred_element_type=jnp.float32) + bor[...]
      o_ref[...] = 1.0 / (1.0 + jnp.exp(-z))

  return pl.pallas_call(
      body,
      grid=(_NB3,),
      in_specs=[
          pl.BlockSpec((_RB3, 2 * _H), lambda j: (j, 0)),
          pl.BlockSpec((_RB3, 2 * _H), lambda j: (half + j, 0)),
          pl.BlockSpec((_RB3, 2 * _G), lambda j: (j, 0)),
          pl.BlockSpec((_RB3, 2 * _G), lambda j: (half + j, 0)),
          pl.BlockSpec((_RB3, 2 * _H), lambda j: (j, 0)),
          pl.BlockSpec((_RB3, 2 * _H), lambda j: (half + j, 0)),
          pl.BlockSpec((_RB3, 2 * _G), lambda j: (j, 0)),
          pl.BlockSpec((_RB3, 2 * _G), lambda j: (half + j, 0)),
          pl.BlockSpec((2 * _H, 2 * _HH), lambda j: (0, 0)),
          pl.BlockSpec((2 * _H, 2 * _HH), lambda j: (0, 0)),
          pl.BlockSpec((1, 2 * _HH), lambda j: (0, 0)),
          pl.BlockSpec((1, 2 * _HH), lambda j: (0, 0)),
          pl.BlockSpec((_H, _H4), lambda j: (0, 0)),
          pl.BlockSpec((1, _H4), lambda j: (0, 0)),
          pl.BlockSpec((_H4, 1), lambda j: (0, 0)),
          pl.BlockSpec((1, 1), lambda j: (0, 0)),
      ],
      out_specs=pl.BlockSpec((1, 1), lambda j: (0, 0)),
      out_shape=jax.ShapeDtypeStruct((1, 1), jnp.float32),
      scratch_shapes=[
          pltpu.VMEM((1, _HH), jnp.float32),
          pltpu.VMEM((1, _HH), jnp.float32),
      ],
  )(a1, a1, c1, c1, a2, a2, c2, c2, Wbd1, Wbd2, bb1, bb2, Wd, bd, Wo, bo)


def _blockdiag(W):
  din, dout = W.shape
  z = jnp.zeros((din, dout), W.dtype)
  return jnp.concatenate(
      [jnp.concatenate([W, z], axis=1), jnp.concatenate([z, W], axis=1)],
      axis=0)


def kernel(x1, edge_index1, x2, edge_index2, W1a, b1a, W1b, b1b,
           W2a, b2a, W2b, b2b, Wd, bd, Wo, bo):
  z_h = jnp.zeros((_ZR, _H), jnp.float32)
  z_c = jnp.zeros((_ZR, _G), jnp.float32)
  ones_g = jnp.ones((_KC, _G), jnp.float32)

  h1, src1, dst1 = _stage1(x1.reshape(_N // 2, 2 * _D), _blockdiag(W1a),
                           jnp.tile(b1a, 2).reshape(1, 2 * _H), edge_index1)
  a1_out, c1_out = _sc_scatter(h1.reshape(_N, _H), src1, dst1,
                               z_h, z_c, ones_g)

  h2, src2, dst2 = _stage1(x2.reshape(_N // 2, 2 * _D), _blockdiag(W2a),
                           jnp.tile(b2a, 2).reshape(1, 2 * _H), edge_index2)
  a2_out, c2_out = _sc_scatter(h2.reshape(_N, _H), src2, dst2,
                               z_h, z_c, ones_g,
                               dep=c1_out[:_ZR, :_G])

  return _stage3(a1_out.reshape(_NP, 2 * _H), c1_out.reshape(_NP, 2 * _G),
                 a2_out.reshape(_NP, 2 * _H), c2_out.reshape(_NP, 2 * _G),
                 _blockdiag(W1b), _blockdiag(W2b),
                 jnp.tile(b1b, 2).reshape(1, 2 * _HH),
                 jnp.tile(b2b, 2).reshape(1, 2 * _HH),
                 Wd, bd.reshape(1, _H4), Wo.reshape(_H4, 1),
                 bo.reshape(1, 1))
